# Initial kernel scaffold; baseline (speedup 1.0000x reference)
import jax
import jax.numpy as jnp
from jax.experimental import pallas as pl


def kernel(edge_index, emb_weight, W1, b1, W2, b2):
    raise NotImplementedError("write your pallas kernel here")


# speedup vs baseline: 7.3329x; 7.3329x over previous
import functools

import jax
import jax.numpy as jnp
from jax import lax
from jax.experimental import pallas as pl
from jax.experimental.pallas import tpu as pltpu
from jax.experimental.pallas import tpu_sc as plsc

N = 10000
NP = 10112
D = 256
DH = 128
E = 160000
EP = 163840
TYPE_NUM = 8000
CHUNK = 128
ROWS_PER_TILE = NP // 16

_f32 = jnp.float32


def _sc_deg_body(dst_hbm, ones_hbm, zeros_hbm, out_hbm, ones_v, idx_v, acc):
    c = lax.axis_index("c")
    s = lax.axis_index("s")
    w = c * 16 + s
    pltpu.sync_copy(ones_hbm, ones_v)
    pltpu.sync_copy(dst_hbm.at[w], idx_v)
    pltpu.sync_copy(zeros_hbm, acc.at[pl.ds(s * ROWS_PER_TILE, ROWS_PER_TILE)])
    plsc.subcore_barrier()

    def step(g, carry):
        pltpu.sync_copy(ones_v, acc.at[idx_v.at[g]], add=True)
        return carry

    lax.fori_loop(0, 40, step, 0)
    plsc.subcore_barrier()
    pltpu.sync_copy(acc.at[pl.ds(s * ROWS_PER_TILE, ROWS_PER_TILE)],
                    out_hbm.at[c, pl.ds(s * ROWS_PER_TILE, ROWS_PER_TILE)])


def _sc_deg(dst3d, ones, zeros128):
    return pl.kernel(
        _sc_deg_body,
        out_type=jax.ShapeDtypeStruct((2, NP, DH), _f32),
        mesh=plsc.VectorSubcoreMesh(core_axis_name="c", subcore_axis_name="s"),
        scratch_types=[
            pltpu.VMEM((CHUNK, DH), _f32),
            pltpu.VMEM((40, CHUNK), jnp.int32),
            pltpu.VMEM_SHARED((NP, DH), _f32),
        ],
    )(dst3d, ones, zeros128)


def _sc_msg_body(z_hbm, src_hbm, dst_hbm, zeros_hbm, out_hbm,
                 src_v, dst_v, rows_v, acc, sem):
    c = lax.axis_index("c")
    s = lax.axis_index("s")
    pltpu.sync_copy(src_hbm.at[s], src_v)
    pltpu.sync_copy(dst_hbm.at[s], dst_v)
    pltpu.sync_copy(zeros_hbm, acc.at[pl.ds(s * ROWS_PER_TILE, ROWS_PER_TILE)])
    plsc.subcore_barrier()

    def step(g, carry):
        pltpu.async_copy(z_hbm.at[c].at[src_v.at[g]], rows_v, sem).wait()
        pltpu.sync_copy(rows_v, acc.at[dst_v.at[g]], add=True)
        return carry

    lax.fori_loop(0, 80, step, 0)
    plsc.subcore_barrier()
    pltpu.sync_copy(acc.at[pl.ds(s * ROWS_PER_TILE, ROWS_PER_TILE)],
                    out_hbm.at[c, pl.ds(s * ROWS_PER_TILE, ROWS_PER_TILE)])


def _sc_msg(z_ext, src3d, dst3d, zeros128):
    return pl.kernel(
        _sc_msg_body,
        out_type=jax.ShapeDtypeStruct((2, NP, DH), _f32),
        mesh=plsc.VectorSubcoreMesh(core_axis_name="c", subcore_axis_name="s"),
        scratch_types=[
            pltpu.VMEM((80, CHUNK), jnp.int32),
            pltpu.VMEM((80, CHUNK), jnp.int32),
            pltpu.VMEM((CHUNK, DH), _f32),
            pltpu.VMEM_SHARED((NP, DH), _f32),
            pltpu.SemaphoreType.DMA,
        ],
    )(z_ext, src3d, dst3d, zeros128)


BLK = NP // 4


def _tc_first_body(x_ref, w_ref, p0_ref, p1_ref, z_ref, dinv_ref):
    dinv = lax.rsqrt(p0_ref[...] + p1_ref[...] + 1.0)
    z = jnp.dot(x_ref[...], w_ref[...], preferred_element_type=_f32) * dinv
    z_ref[0, :, :] = z[:, :DH]
    z_ref[1, :, :] = z[:, DH:]
    dinv_ref[...] = dinv


def _tc_first(x_pad, w1, p0, p1):
    return pl.pallas_call(
        _tc_first_body,
        grid=(4,),
        in_specs=[
            pl.BlockSpec((BLK, D), lambda i: (i, 0)),
            pl.BlockSpec((D, D), lambda i: (0, 0)),
            pl.BlockSpec((BLK, 1), lambda i: (i, 0)),
            pl.BlockSpec((BLK, 1), lambda i: (i, 0)),
        ],
        out_specs=[
            pl.BlockSpec((2, BLK, DH), lambda i: (0, i, 0)),
            pl.BlockSpec((BLK, 1), lambda i: (i, 0)),
        ],
        out_shape=[
            jax.ShapeDtypeStruct((2, NP, DH), _f32),
            jax.ShapeDtypeStruct((NP, 1), _f32),
        ],
    )(x_pad, w1, p0, p1)


def _gelu(x):
    return 0.5 * x * (1.0 + lax.erf(x * 0.7071067811865476))


def _tc_mid_body(s_ref, z_ref, dinv_ref, b_ref, w_ref, z2_ref):
    dinv = dinv_ref[...]
    sfull = jnp.concatenate([s_ref[0], s_ref[1]], axis=1)
    zfull = jnp.concatenate([z_ref[0], z_ref[1]], axis=1)
    h = _gelu(dinv * (sfull + zfull) + b_ref[...])
    z2 = jnp.dot(h, w_ref[...], preferred_element_type=_f32) * dinv
    z2_ref[0, :, :] = z2[:, :DH]
    z2_ref[1, :, :] = z2[:, DH:]


def _tc_mid(s1, z1, dinv, b1, w2):
    return pl.pallas_call(
        _tc_mid_body,
        grid=(4,),
        in_specs=[
            pl.BlockSpec((2, BLK, DH), lambda i: (0, i, 0)),
            pl.BlockSpec((2, BLK, DH), lambda i: (0, i, 0)),
            pl.BlockSpec((BLK, 1), lambda i: (i, 0)),
            pl.BlockSpec((1, D), lambda i: (0, 0)),
            pl.BlockSpec((D, D), lambda i: (0, 0)),
        ],
        out_specs=pl.BlockSpec((2, BLK, DH), lambda i: (0, i, 0)),
        out_shape=jax.ShapeDtypeStruct((2, NP, DH), _f32),
    )(s1, z1, dinv, b1, w2)


def _tc_last_body(s_ref, z_ref, dinv_ref, b_ref, out_ref):
    sfull = jnp.concatenate([s_ref[0], s_ref[1]], axis=1)
    zfull = jnp.concatenate([z_ref[0], z_ref[1]], axis=1)
    out_ref[...] = _gelu(dinv_ref[...] * (sfull + zfull) + b_ref[...])


def _tc_last(s2, z2, dinv, b2):
    return pl.pallas_call(
        _tc_last_body,
        grid=(4,),
        in_specs=[
            pl.BlockSpec((2, BLK, DH), lambda i: (0, i, 0)),
            pl.BlockSpec((2, BLK, DH), lambda i: (0, i, 0)),
            pl.BlockSpec((BLK, 1), lambda i: (i, 0)),
            pl.BlockSpec((1, D), lambda i: (0, 0)),
        ],
        out_specs=pl.BlockSpec((BLK, D), lambda i: (i, 0)),
        out_shape=jax.ShapeDtypeStruct((NP, D), _f32),
    )(s2, z2, dinv, b2)


@jax.jit
def kernel(edge_index, emb_weight, W1, b1, W2, b2):
    src = edge_index[0]
    dst = edge_index[1]
    pad = jnp.full((EP - E,), N, dtype=jnp.int32)
    src_p = jnp.concatenate([src, pad])
    dst_p = jnp.concatenate([dst, pad])
    src3d = src_p.reshape(16, 80, CHUNK)
    dst3d_msg = dst_p.reshape(16, 80, CHUNK)
    dst3d_deg = dst_p.reshape(32, 40, CHUNK)

    ones128 = jnp.ones((CHUNK, DH), dtype=_f32)
    zeros128 = jnp.zeros((ROWS_PER_TILE, DH), dtype=_f32)

    x_pad = jnp.zeros((NP, D), dtype=_f32).at[:N].set(emb_weight)
    b1r = b1.reshape(1, D)
    b2r = b2.reshape(1, D)

    degp = _sc_deg(dst3d_deg, ones128, zeros128)
    p0 = degp[0, :, 0:1]
    p1 = degp[1, :, 0:1]

    z1, dinv = _tc_first(x_pad, W1, p0, p1)
    s1 = _sc_msg(z1, src3d, dst3d_msg, zeros128)
    z2 = _tc_mid(s1, z1, dinv, b1r, W2)
    s2 = _sc_msg(z2, src3d, dst3d_msg, zeros128)
    out = _tc_last(s2, z2, dinv, b2r)
    return (out[:TYPE_NUM], out[TYPE_NUM:N])


# speedup vs baseline: 7.9093x; 1.0786x over previous
import functools

import jax
import jax.numpy as jnp
from jax import lax
from jax.experimental import pallas as pl
from jax.experimental.pallas import tpu as pltpu
from jax.experimental.pallas import tpu_sc as plsc

N = 10000
NP = 10112
D = 256
DH = 128
E = 160000
EP = 163840
TYPE_NUM = 8000
CHUNK = 128
ROWS_PER_TILE = NP // 16

_f32 = jnp.float32


def _sc_deg_body(dst_hbm, ones_hbm, zeros_hbm, out_hbm, ones_v, idx_v, acc):
    c = lax.axis_index("c")
    s = lax.axis_index("s")
    w = c * 16 + s
    pltpu.sync_copy(ones_hbm, ones_v)
    pltpu.sync_copy(dst_hbm.at[w], idx_v)
    pltpu.sync_copy(zeros_hbm, acc.at[pl.ds(s * ROWS_PER_TILE, ROWS_PER_TILE)])
    plsc.subcore_barrier()

    def step(g, carry):
        pltpu.sync_copy(ones_v, acc.at[idx_v.at[g]], add=True)
        return carry

    lax.fori_loop(0, 40, step, 0)
    plsc.subcore_barrier()
    pltpu.sync_copy(acc.at[pl.ds(s * ROWS_PER_TILE, ROWS_PER_TILE)],
                    out_hbm.at[c, pl.ds(s * ROWS_PER_TILE, ROWS_PER_TILE)])


def _sc_deg(dst3d, ones, zeros128):
    return pl.kernel(
        _sc_deg_body,
        out_type=jax.ShapeDtypeStruct((2, NP, DH), _f32),
        mesh=plsc.VectorSubcoreMesh(core_axis_name="c", subcore_axis_name="s"),
        scratch_types=[
            pltpu.VMEM((CHUNK, DH), _f32),
            pltpu.VMEM((40, CHUNK), jnp.int32),
            pltpu.VMEM_SHARED((NP, DH), _f32),
        ],
    )(dst3d, ones, zeros128)


def _sc_msg_body(z_hbm, src_hbm, dst_hbm, zeros_hbm, out_hbm,
                 src_v, dst_v, rows0, rows1, acc, sem0, sem1):
    c = lax.axis_index("c")
    s = lax.axis_index("s")
    for j in range(4):
        pltpu.sync_copy(zeros_hbm,
                        acc.at[pl.ds(s * ROWS_PER_TILE + j * 152, 152)])
    pltpu.sync_copy(zeros_hbm.at[pl.ds(0, 24)],
                    acc.at[pl.ds(s * ROWS_PER_TILE + 608, 24)])
    plsc.subcore_barrier()

    def run_stage(base):
        pltpu.sync_copy(src_hbm.at[s].at[pl.ds(base, 40)], src_v)
        pltpu.sync_copy(dst_hbm.at[s].at[pl.ds(base, 40)], dst_v)
        pltpu.async_copy(z_hbm.at[c].at[src_v.at[0]], rows0, sem0)

        def step(i, carry):
            g0 = 2 * i
            pltpu.async_copy(z_hbm.at[c].at[src_v.at[g0 + 1]], rows1, sem1)
            pltpu.make_async_copy(z_hbm.at[c].at[src_v.at[0]], rows0, sem0).wait()
            pltpu.sync_copy(rows0, acc.at[dst_v.at[g0]], add=True)
            g2 = jnp.minimum(g0 + 2, 39)
            pltpu.async_copy(z_hbm.at[c].at[src_v.at[g2]], rows0, sem0)
            pltpu.make_async_copy(z_hbm.at[c].at[src_v.at[0]], rows1, sem1).wait()
            pltpu.sync_copy(rows1, acc.at[dst_v.at[g0 + 1]], add=True)
            return carry

        lax.fori_loop(0, 20, step, 0)
        pltpu.make_async_copy(z_hbm.at[c].at[src_v.at[0]], rows0, sem0).wait()

    run_stage(0)
    run_stage(40)
    plsc.subcore_barrier()
    pltpu.sync_copy(acc.at[pl.ds(s * ROWS_PER_TILE, ROWS_PER_TILE)],
                    out_hbm.at[c, pl.ds(s * ROWS_PER_TILE, ROWS_PER_TILE)])


def _sc_msg(z_ext, src3d, dst3d, zeros128):
    return pl.kernel(
        _sc_msg_body,
        out_type=jax.ShapeDtypeStruct((2, NP, DH), _f32),
        mesh=plsc.VectorSubcoreMesh(core_axis_name="c", subcore_axis_name="s"),
        scratch_types=[
            pltpu.VMEM((40, CHUNK), jnp.int32),
            pltpu.VMEM((40, CHUNK), jnp.int32),
            pltpu.VMEM((CHUNK, DH), _f32),
            pltpu.VMEM((CHUNK, DH), _f32),
            pltpu.VMEM_SHARED((NP, DH), _f32),
            pltpu.SemaphoreType.DMA,
            pltpu.SemaphoreType.DMA,
        ],
    )(z_ext, src3d, dst3d, zeros128)


BLK = NP // 4


def _tc_first_body(x_ref, w_ref, p0_ref, p1_ref, z_ref, dinv_ref):
    dinv = lax.rsqrt(p0_ref[...] + p1_ref[...] + 1.0)
    z = jnp.dot(x_ref[...], w_ref[...], preferred_element_type=_f32) * dinv
    z_ref[0, :, :] = z[:, :DH]
    z_ref[1, :, :] = z[:, DH:]
    dinv_ref[...] = dinv


def _tc_first(x_pad, w1, p0, p1):
    return pl.pallas_call(
        _tc_first_body,
        grid=(4,),
        in_specs=[
            pl.BlockSpec((BLK, D), lambda i: (i, 0)),
            pl.BlockSpec((D, D), lambda i: (0, 0)),
            pl.BlockSpec((BLK, 1), lambda i: (i, 0)),
            pl.BlockSpec((BLK, 1), lambda i: (i, 0)),
        ],
        out_specs=[
            pl.BlockSpec((2, BLK, DH), lambda i: (0, i, 0)),
            pl.BlockSpec((BLK, 1), lambda i: (i, 0)),
        ],
        out_shape=[
            jax.ShapeDtypeStruct((2, NP, DH), _f32),
            jax.ShapeDtypeStruct((NP, 1), _f32),
        ],
    )(x_pad, w1, p0, p1)


def _gelu(x):
    return 0.5 * x * (1.0 + lax.erf(x * 0.7071067811865476))


def _tc_mid_body(s_ref, z_ref, dinv_ref, b_ref, w_ref, z2_ref):
    dinv = dinv_ref[...]
    sfull = jnp.concatenate([s_ref[0], s_ref[1]], axis=1)
    zfull = jnp.concatenate([z_ref[0], z_ref[1]], axis=1)
    h = _gelu(dinv * (sfull + zfull) + b_ref[...])
    z2 = jnp.dot(h, w_ref[...], preferred_element_type=_f32) * dinv
    z2_ref[0, :, :] = z2[:, :DH]
    z2_ref[1, :, :] = z2[:, DH:]


def _tc_mid(s1, z1, dinv, b1, w2):
    return pl.pallas_call(
        _tc_mid_body,
        grid=(4,),
        in_specs=[
            pl.BlockSpec((2, BLK, DH), lambda i: (0, i, 0)),
            pl.BlockSpec((2, BLK, DH), lambda i: (0, i, 0)),
            pl.BlockSpec((BLK, 1), lambda i: (i, 0)),
            pl.BlockSpec((1, D), lambda i: (0, 0)),
            pl.BlockSpec((D, D), lambda i: (0, 0)),
        ],
        out_specs=pl.BlockSpec((2, BLK, DH), lambda i: (0, i, 0)),
        out_shape=jax.ShapeDtypeStruct((2, NP, DH), _f32),
    )(s1, z1, dinv, b1, w2)


def _tc_last_body(s_ref, z_ref, dinv_ref, b_ref, out_ref):
    sfull = jnp.concatenate([s_ref[0], s_ref[1]], axis=1)
    zfull = jnp.concatenate([z_ref[0], z_ref[1]], axis=1)
    out_ref[...] = _gelu(dinv_ref[...] * (sfull + zfull) + b_ref[...])


def _tc_last(s2, z2, dinv, b2):
    return pl.pallas_call(
        _tc_last_body,
        grid=(4,),
        in_specs=[
            pl.BlockSpec((2, BLK, DH), lambda i: (0, i, 0)),
            pl.BlockSpec((2, BLK, DH), lambda i: (0, i, 0)),
            pl.BlockSpec((BLK, 1), lambda i: (i, 0)),
            pl.BlockSpec((1, D), lambda i: (0, 0)),
        ],
        out_specs=pl.BlockSpec((BLK, D), lambda i: (i, 0)),
        out_shape=jax.ShapeDtypeStruct((NP, D), _f32),
    )(s2, z2, dinv, b2)


@jax.jit
def kernel(edge_index, emb_weight, W1, b1, W2, b2):
    src = edge_index[0]
    dst = edge_index[1]
    pad = jnp.full((EP - E,), N, dtype=jnp.int32)
    src_p = jnp.concatenate([src, pad])
    dst_p = jnp.concatenate([dst, pad])
    src3d = src_p.reshape(16, 80, CHUNK)
    dst3d_msg = dst_p.reshape(16, 80, CHUNK)
    dst3d_deg = dst_p.reshape(32, 40, CHUNK)

    ones128 = jnp.ones((CHUNK, DH), dtype=_f32)
    zeros128 = jnp.zeros((ROWS_PER_TILE, DH), dtype=_f32)
    zeros152 = jnp.zeros((152, DH), dtype=_f32)

    x_pad = jnp.zeros((NP, D), dtype=_f32).at[:N].set(emb_weight)
    b1r = b1.reshape(1, D)
    b2r = b2.reshape(1, D)

    degp = _sc_deg(dst3d_deg, ones128, zeros128)
    p0 = degp[0, :, 0:1]
    p1 = degp[1, :, 0:1]

    z1, dinv = _tc_first(x_pad, W1, p0, p1)
    s1 = _sc_msg(z1, src3d, dst3d_msg, zeros152)
    z2 = _tc_mid(s1, z1, dinv, b1r, W2)
    s2 = _sc_msg(z2, src3d, dst3d_msg, zeros152)
    out = _tc_last(s2, z2, dinv, b2r)
    return (out[:TYPE_NUM], out[TYPE_NUM:N])


# speedup vs baseline: 7.9485x; 1.0050x over previous
import functools

import jax
import jax.numpy as jnp
from jax import lax
from jax.experimental import pallas as pl
from jax.experimental.pallas import tpu as pltpu
from jax.experimental.pallas import tpu_sc as plsc

N = 10000
NP = 10112
D = 256
DH = 128
E = 160000
EP = 163840
TYPE_NUM = 8000
DEG_CHUNK = 128
CHUNK = 64
ROWS_PER_TILE = NP // 16

_f32 = jnp.float32


def _sc_deg_body(dst_hbm, ones_hbm, zeros_hbm, out_hbm, ones_v, idx_v, acc):
    c = lax.axis_index("c")
    s = lax.axis_index("s")
    w = c * 16 + s
    pltpu.sync_copy(ones_hbm, ones_v)
    pltpu.sync_copy(dst_hbm.at[w], idx_v)
    pltpu.sync_copy(zeros_hbm, acc.at[pl.ds(s * ROWS_PER_TILE, ROWS_PER_TILE)])
    plsc.subcore_barrier()

    def step(g, carry):
        pltpu.sync_copy(ones_v, acc.at[idx_v.at[g]], add=True)
        return carry

    lax.fori_loop(0, 40, step, 0)
    plsc.subcore_barrier()
    pltpu.sync_copy(acc.at[pl.ds(s * ROWS_PER_TILE, ROWS_PER_TILE)],
                    out_hbm.at[c, pl.ds(s * ROWS_PER_TILE, ROWS_PER_TILE)])


def _sc_deg(dst3d, ones, zeros128):
    return pl.kernel(
        _sc_deg_body,
        out_type=jax.ShapeDtypeStruct((2, NP, DH), _f32),
        mesh=plsc.VectorSubcoreMesh(core_axis_name="c", subcore_axis_name="s"),
        scratch_types=[
            pltpu.VMEM((DEG_CHUNK, DH), _f32),
            pltpu.VMEM((40, DEG_CHUNK), jnp.int32),
            pltpu.VMEM_SHARED((NP, DH), _f32),
        ],
    )(dst3d, ones, zeros128)


def _sc_msg_body(z_hbm, src_hbm, dst_hbm, zeros_hbm, out_hbm,
                 src_v, dst_v, r0, r1, r2, r3, acc,
                 g0, g1, g2, g3, s0, s1, s2, s3):
    c = lax.axis_index("c")
    s = lax.axis_index("s")
    rows = (r0, r1, r2, r3)
    gsem = (g0, g1, g2, g3)
    ssem = (s0, s1, s2, s3)
    for j in range(4):
        pltpu.sync_copy(zeros_hbm,
                        acc.at[pl.ds(s * ROWS_PER_TILE + j * 152, 152)])
    pltpu.sync_copy(zeros_hbm.at[pl.ds(0, 24)],
                    acc.at[pl.ds(s * ROWS_PER_TILE + 608, 24)])
    plsc.subcore_barrier()

    def run_stage(base):
        pltpu.sync_copy(src_hbm.at[s].at[pl.ds(base, 40)], src_v)
        pltpu.sync_copy(dst_hbm.at[s].at[pl.ds(base, 40)], dst_v)
        for b in range(4):
            pltpu.async_copy(z_hbm.at[c].at[src_v.at[b]], rows[b], gsem[b])

        def step(i, carry):
            g = 4 * i
            for b in range(4):
                pltpu.make_async_copy(z_hbm.at[c].at[src_v.at[0]], rows[b],
                                      gsem[b]).wait()
                pltpu.async_copy(rows[b], acc.at[dst_v.at[g + b]], ssem[b],
                                 add=True)
            for b in range(4):
                pltpu.make_async_copy(rows[b], acc.at[dst_v.at[0]],
                                      ssem[b]).wait()
                pltpu.async_copy(z_hbm.at[c].at[src_v.at[g + 4 + b]], rows[b],
                                 gsem[b])
            return carry

        lax.fori_loop(0, 9, step, 0)
        for b in range(4):
            pltpu.make_async_copy(z_hbm.at[c].at[src_v.at[0]], rows[b],
                                  gsem[b]).wait()
            pltpu.async_copy(rows[b], acc.at[dst_v.at[36 + b]], ssem[b],
                             add=True)
        for b in range(4):
            pltpu.make_async_copy(rows[b], acc.at[dst_v.at[0]], ssem[b]).wait()

    run_stage(0)
    run_stage(40)
    run_stage(80)
    run_stage(120)
    plsc.subcore_barrier()
    pltpu.sync_copy(acc.at[pl.ds(s * ROWS_PER_TILE, ROWS_PER_TILE)],
                    out_hbm.at[c, pl.ds(s * ROWS_PER_TILE, ROWS_PER_TILE)])


def _sc_msg(z_ext, src3d, dst3d, zeros128):
    return pl.kernel(
        _sc_msg_body,
        out_type=jax.ShapeDtypeStruct((2, NP, DH), _f32),
        mesh=plsc.VectorSubcoreMesh(core_axis_name="c", subcore_axis_name="s"),
        scratch_types=[
            pltpu.VMEM((40, CHUNK), jnp.int32),
            pltpu.VMEM((40, CHUNK), jnp.int32),
            pltpu.VMEM((CHUNK, DH), _f32),
            pltpu.VMEM((CHUNK, DH), _f32),
            pltpu.VMEM((CHUNK, DH), _f32),
            pltpu.VMEM((CHUNK, DH), _f32),
            pltpu.VMEM_SHARED((NP, DH), _f32),
            pltpu.SemaphoreType.DMA,
            pltpu.SemaphoreType.DMA,
            pltpu.SemaphoreType.DMA,
            pltpu.SemaphoreType.DMA,
            pltpu.SemaphoreType.DMA,
            pltpu.SemaphoreType.DMA,
            pltpu.SemaphoreType.DMA,
            pltpu.SemaphoreType.DMA,
        ],
    )(z_ext, src3d, dst3d, zeros128)


BLK = NP // 4


def _tc_first_body(x_ref, w_ref, p0_ref, p1_ref, z_ref, dinv_ref):
    dinv = lax.rsqrt(p0_ref[...] + p1_ref[...] + 1.0)
    z = jnp.dot(x_ref[...], w_ref[...], preferred_element_type=_f32) * dinv
    z_ref[0, :, :] = z[:, :DH]
    z_ref[1, :, :] = z[:, DH:]
    dinv_ref[...] = dinv


def _tc_first(x_pad, w1, p0, p1):
    return pl.pallas_call(
        _tc_first_body,
        grid=(4,),
        in_specs=[
            pl.BlockSpec((BLK, D), lambda i: (i, 0)),
            pl.BlockSpec((D, D), lambda i: (0, 0)),
            pl.BlockSpec((BLK, 1), lambda i: (i, 0)),
            pl.BlockSpec((BLK, 1), lambda i: (i, 0)),
        ],
        out_specs=[
            pl.BlockSpec((2, BLK, DH), lambda i: (0, i, 0)),
            pl.BlockSpec((BLK, 1), lambda i: (i, 0)),
        ],
        out_shape=[
            jax.ShapeDtypeStruct((2, NP, DH), _f32),
            jax.ShapeDtypeStruct((NP, 1), _f32),
        ],
    )(x_pad, w1, p0, p1)


def _gelu(x):
    return 0.5 * x * (1.0 + lax.erf(x * 0.7071067811865476))


def _tc_mid_body(s_ref, z_ref, dinv_ref, b_ref, w_ref, z2_ref):
    dinv = dinv_ref[...]
    sfull = jnp.concatenate([s_ref[0], s_ref[1]], axis=1)
    zfull = jnp.concatenate([z_ref[0], z_ref[1]], axis=1)
    h = _gelu(dinv * (sfull + zfull) + b_ref[...])
    z2 = jnp.dot(h, w_ref[...], preferred_element_type=_f32) * dinv
    z2_ref[0, :, :] = z2[:, :DH]
    z2_ref[1, :, :] = z2[:, DH:]


def _tc_mid(s1, z1, dinv, b1, w2):
    return pl.pallas_call(
        _tc_mid_body,
        grid=(4,),
        in_specs=[
            pl.BlockSpec((2, BLK, DH), lambda i: (0, i, 0)),
            pl.BlockSpec((2, BLK, DH), lambda i: (0, i, 0)),
            pl.BlockSpec((BLK, 1), lambda i: (i, 0)),
            pl.BlockSpec((1, D), lambda i: (0, 0)),
            pl.BlockSpec((D, D), lambda i: (0, 0)),
        ],
        out_specs=pl.BlockSpec((2, BLK, DH), lambda i: (0, i, 0)),
        out_shape=jax.ShapeDtypeStruct((2, NP, DH), _f32),
    )(s1, z1, dinv, b1, w2)


def _tc_last_body(s_ref, z_ref, dinv_ref, b_ref, out_ref):
    sfull = jnp.concatenate([s_ref[0], s_ref[1]], axis=1)
    zfull = jnp.concatenate([z_ref[0], z_ref[1]], axis=1)
    out_ref[...] = _gelu(dinv_ref[...] * (sfull + zfull) + b_ref[...])


def _tc_last(s2, z2, dinv, b2):
    return pl.pallas_call(
        _tc_last_body,
        grid=(4,),
        in_specs=[
            pl.BlockSpec((2, BLK, DH), lambda i: (0, i, 0)),
            pl.BlockSpec((2, BLK, DH), lambda i: (0, i, 0)),
            pl.BlockSpec((BLK, 1), lambda i: (i, 0)),
            pl.BlockSpec((1, D), lambda i: (0, 0)),
        ],
        out_specs=pl.BlockSpec((BLK, D), lambda i: (i, 0)),
        out_shape=jax.ShapeDtypeStruct((NP, D), _f32),
    )(s2, z2, dinv, b2)


@jax.jit
def kernel(edge_index, emb_weight, W1, b1, W2, b2):
    src = edge_index[0]
    dst = edge_index[1]
    pad = jnp.full((EP - E,), N, dtype=jnp.int32)
    src_p = jnp.concatenate([src, pad])
    dst_p = jnp.concatenate([dst, pad])
    src3d = src_p.reshape(16, 160, CHUNK)
    dst3d_msg = dst_p.reshape(16, 160, CHUNK)
    dst3d_deg = dst_p.reshape(32, 40, DEG_CHUNK)

    ones128 = jnp.ones((DEG_CHUNK, DH), dtype=_f32)
    zeros128 = jnp.zeros((ROWS_PER_TILE, DH), dtype=_f32)
    zeros152 = jnp.zeros((152, DH), dtype=_f32)

    x_pad = jnp.zeros((NP, D), dtype=_f32).at[:N].set(emb_weight)
    b1r = b1.reshape(1, D)
    b2r = b2.reshape(1, D)

    degp = _sc_deg(dst3d_deg, ones128, zeros128)
    p0 = degp[0, :, 0:1]
    p1 = degp[1, :, 0:1]

    z1, dinv = _tc_first(x_pad, W1, p0, p1)
    s1 = _sc_msg(z1, src3d, dst3d_msg, zeros152)
    z2 = _tc_mid(s1, z1, dinv, b1r, W2)
    s2 = _sc_msg(z2, src3d, dst3d_msg, zeros152)
    out = _tc_last(s2, z2, dinv, b2r)
    return (out[:TYPE_NUM], out[TYPE_NUM:N])
